# speedup vs baseline: 1.1180x; 1.1180x over previous
import functools

import jax
import jax.numpy as jnp
from jax import lax
from jax.experimental import pallas as pl
from jax.experimental.pallas import tpu as pltpu
from jax.experimental.pallas import tpu_sc as plsc

N_G1 = 110000
N_G2 = 100000
D = 128
H = 64
C = 8
R = 8
B = 4
E = 320000

ROWS_BLK = 2000
N_BLKS = N_G2 // ROWS_BLK


def _tc1_body(cnt_a_ref, cnt_b_ref, invn_ref):
    cnt = cnt_a_ref[...] + cnt_b_ref[...]
    invn_ref[...] = 1.0 / jnp.maximum(cnt, 1.0)


def _tc1_invnorm(cnt_a, cnt_b):
    shaped = (N_G2 * R // 128, 128)
    return pl.pallas_call(
        _tc1_body,
        out_shape=jax.ShapeDtypeStruct(shaped, jnp.float32),
    )(cnt_a.reshape(shaped), cnt_b.reshape(shaped)).reshape(N_G2 * R)


def _tc2_body(agg_ref, deg_ref, w1a_ref, w1b_ref, hb1_ref, xr_ref):
    x = jnp.maximum(agg_ref[...] / jnp.maximum(deg_ref[...], 1.0), 0.0)
    hb1_ref[...] = jnp.dot(x, w1a_ref[...], preferred_element_type=jnp.float32)
    xr_ref[...] = jnp.dot(x, w1b_ref[...], preferred_element_type=jnp.float32)


def _tc2(aggA, deg, w1a, w1b):
    return pl.pallas_call(
        _tc2_body,
        grid=(N_BLKS,),
        in_specs=[
            pl.BlockSpec((ROWS_BLK, D), lambda i: (i, 0)),
            pl.BlockSpec((ROWS_BLK, 1), lambda i: (i, 0)),
            pl.BlockSpec((D, B * H), lambda i: (0, 0)),
            pl.BlockSpec((D, H), lambda i: (0, 0)),
        ],
        out_specs=[
            pl.BlockSpec((ROWS_BLK, B * H), lambda i: (i, 0)),
            pl.BlockSpec((ROWS_BLK, H), lambda i: (i, 0)),
        ],
        out_shape=[
            jax.ShapeDtypeStruct((N_G2, B * H), jnp.float32),
            jax.ShapeDtypeStruct((N_G2, H), jnp.float32),
        ],
    )(aggA, deg.reshape(N_G2, 1), w1a, w1b)


def _tc3_body(agg_ref, xr_ref, b1_ref, w2a_ref, w2b_ref, hb2_ref, hr_ref):
    h = jnp.maximum(agg_ref[...] + xr_ref[...] + b1_ref[...], 0.0)
    hb2_ref[...] = jnp.dot(h, w2a_ref[...], preferred_element_type=jnp.float32)
    hr_ref[...] = jnp.dot(h, w2b_ref[...], preferred_element_type=jnp.float32)


def _tc3(aggB, xr, bias1, w2a, w2b):
    return pl.pallas_call(
        _tc3_body,
        grid=(N_BLKS,),
        in_specs=[
            pl.BlockSpec((ROWS_BLK, H), lambda i: (i, 0)),
            pl.BlockSpec((ROWS_BLK, H), lambda i: (i, 0)),
            pl.BlockSpec((1, H), lambda i: (0, 0)),
            pl.BlockSpec((H, B * C), lambda i: (0, 0)),
            pl.BlockSpec((H, C), lambda i: (0, 0)),
        ],
        out_specs=[
            pl.BlockSpec((ROWS_BLK, B * C), lambda i: (i, 0)),
            pl.BlockSpec((ROWS_BLK, C), lambda i: (i, 0)),
        ],
        out_shape=[
            jax.ShapeDtypeStruct((N_G2, B * C), jnp.float32),
            jax.ShapeDtypeStruct((N_G2, C), jnp.float32),
        ],
    )(aggB, xr, bias1.reshape(1, H), w2a, w2b)


def _tc4_body(agg_a_ref, agg_b_ref, hr_ref, b2_ref, out_ref):
    z = agg_a_ref[...] + agg_b_ref[...] + hr_ref[...] + b2_ref[...]
    z = z - jnp.max(z, axis=1, keepdims=True)
    ez = jnp.exp(z)
    out_ref[...] = ez / jnp.sum(ez, axis=1, keepdims=True)


def _tc4(aggC_a, aggC_b, hr, bias2):
    return pl.pallas_call(
        _tc4_body,
        grid=(N_BLKS,),
        in_specs=[
            pl.BlockSpec((ROWS_BLK, C), lambda i: (i, 0)),
            pl.BlockSpec((ROWS_BLK, C), lambda i: (i, 0)),
            pl.BlockSpec((ROWS_BLK, C), lambda i: (i, 0)),
            pl.BlockSpec((1, C), lambda i: (0, 0)),
        ],
        out_specs=pl.BlockSpec((ROWS_BLK, C), lambda i: (i, 0)),
        out_shape=jax.ShapeDtypeStruct((N_G2, C), jnp.float32),
    )(aggC_a, aggC_b, hr, bias2.reshape(1, C))


def _jnp_counts(dst2, rel):
    keyid = dst2 * R + rel
    cnt = jax.ops.segment_sum(jnp.ones(E, jnp.float32), keyid,
                              num_segments=N_G2 * R)
    return cnt, jnp.zeros_like(cnt)


def _jnp_deg_agg_g1(src1, dst1, emb):
    m = dst1 < N_G2
    sdst = jnp.where(m, dst1, 0)
    deg = jax.ops.segment_sum(jnp.where(m, 1.0, 0.0), sdst, num_segments=N_G2)
    agg = jax.ops.segment_sum(jnp.where(m[:, None], emb[src1], 0.0), sdst,
                              num_segments=N_G2)
    return agg, deg


def _jnp_edge_w(invn, keyid, rel, comp1, comp2):
    nrm = invn[keyid][:, None]
    return nrm * comp1[rel], nrm * comp2[rel]


def _jnp_edge_pass(table, w, src, dst, width):
    rows = table[src].reshape(E, B, width)
    msg = jnp.einsum('eb,ebh->eh', w, rows)
    agg = jax.ops.segment_sum(msg, dst, num_segments=N_G2)
    return agg, jnp.zeros_like(agg)


def kernel(edge_index_g2, edge_type_g2, edge_index_g1, all_node_embedding,
           basis1, comp1, root1, bias1, basis2, comp2, root2, bias2):
    src1 = edge_index_g1[0]
    dst1 = edge_index_g1[1]
    src2 = edge_index_g2[0]
    dst2 = edge_index_g2[1]
    rel = edge_type_g2
    keyid = dst2 * R + rel

    w1a = basis1.transpose(1, 0, 2).reshape(D, B * H)
    w2a = basis2.transpose(1, 0, 2).reshape(H, B * C)

    cnt_a, cnt_b = _jnp_counts(dst2, rel)
    invn = _tc1_invnorm(cnt_a, cnt_b)

    aggA, deg = _jnp_deg_agg_g1(src1, dst1, all_node_embedding)
    w1, w2 = _jnp_edge_w(invn, keyid, rel, comp1, comp2)

    hb1, xr = _tc2(aggA, deg, w1a, root1)
    aggB_a, aggB_b = _jnp_edge_pass(hb1, w1, src2, dst2, H)
    hb2, hr = _tc3(aggB_a + aggB_b, xr, bias1, w2a, root2)
    aggC_a, aggC_b = _jnp_edge_pass(hb2, w2, src2, dst2, C)
    return _tc4(aggC_a, aggC_b, hr, bias2)


# speedup vs baseline: 1.4078x; 1.2592x over previous
import functools

import jax
import jax.numpy as jnp
from jax import lax
from jax.experimental import pallas as pl
from jax.experimental.pallas import tpu as pltpu
from jax.experimental.pallas import tpu_sc as plsc

N_G1 = 110000
N_G2 = 100000
D = 128
H = 64
C = 8
R = 8
B = 4
E = 320000

ROWS_BLK = 2000
N_BLKS = N_G2 // ROWS_BLK

NC = 2
NS = 16
L = 16
EG = 128
E_PAD = 327680
TILE_E = E_PAD // (NC * NS)
EBLKS = TILE_E // EG
CPAD = 16
N2P = 100352


def _mesh():
    return plsc.VectorSubcoreMesh(core_axis_name="c", subcore_axis_name="s",
                                  num_cores=NC, num_subcores=NS)


def _tc1_body(cnt_a_ref, cnt_b_ref, invn_ref):
    cnt = cnt_a_ref[...] + cnt_b_ref[...]
    invn_ref[...] = 1.0 / jnp.maximum(cnt, 1.0)


def _tc1_invnorm(cnt_a, cnt_b):
    shaped = (N_G2 * R // 128, 128)
    return pl.pallas_call(
        _tc1_body,
        out_shape=jax.ShapeDtypeStruct(shaped, jnp.float32),
    )(cnt_a.reshape(shaped), cnt_b.reshape(shaped)).reshape(N_G2 * R)


def _tc2_body(agg_ref, deg_ref, w1a_ref, w1b_ref, hb1_ref, xr_ref):
    x = jnp.maximum(agg_ref[...] / jnp.maximum(deg_ref[...], 1.0), 0.0)
    hb1_ref[...] = jnp.dot(x, w1a_ref[...], preferred_element_type=jnp.float32)
    xr_ref[...] = jnp.dot(x, w1b_ref[...], preferred_element_type=jnp.float32)


def _tc2(aggA, deg, w1a, w1b):
    return pl.pallas_call(
        _tc2_body,
        grid=(N_BLKS,),
        in_specs=[
            pl.BlockSpec((ROWS_BLK, D), lambda i: (i, 0)),
            pl.BlockSpec((ROWS_BLK, 1), lambda i: (i, 0)),
            pl.BlockSpec((D, B * H), lambda i: (0, 0)),
            pl.BlockSpec((D, H), lambda i: (0, 0)),
        ],
        out_specs=[
            pl.BlockSpec((ROWS_BLK, B * H), lambda i: (i, 0)),
            pl.BlockSpec((ROWS_BLK, H), lambda i: (i, 0)),
        ],
        out_shape=[
            jax.ShapeDtypeStruct((N_G2, B * H), jnp.float32),
            jax.ShapeDtypeStruct((N_G2, H), jnp.float32),
        ],
    )(aggA, deg.reshape(N_G2, 1), w1a, w1b)


def _tc3_body(agg_ref, xr_ref, b1_ref, w2r_ref, w2b_ref, hb2r_ref, hr_ref):
    h = jnp.maximum(agg_ref[...] + xr_ref[...] + b1_ref[...], 0.0)
    for r in range(R):
        hb2r_ref[r] = jnp.dot(h, w2r_ref[r],
                              preferred_element_type=jnp.float32)
    hr_ref[...] = jnp.dot(h, w2b_ref[...], preferred_element_type=jnp.float32)


def _tc3(aggB, xr, bias1, w2r, w2b):
    return pl.pallas_call(
        _tc3_body,
        grid=(N_BLKS,),
        in_specs=[
            pl.BlockSpec((ROWS_BLK, H), lambda i: (i, 0)),
            pl.BlockSpec((ROWS_BLK, H), lambda i: (i, 0)),
            pl.BlockSpec((1, H), lambda i: (0, 0)),
            pl.BlockSpec((R, H, CPAD), lambda i: (0, 0, 0)),
            pl.BlockSpec((H, C), lambda i: (0, 0)),
        ],
        out_specs=[
            pl.BlockSpec((R, ROWS_BLK, CPAD), lambda i: (0, i, 0)),
            pl.BlockSpec((ROWS_BLK, C), lambda i: (i, 0)),
        ],
        out_shape=[
            jax.ShapeDtypeStruct((R, N_G2, CPAD), jnp.float32),
            jax.ShapeDtypeStruct((N_G2, C), jnp.float32),
        ],
    )(aggB, xr, bias1.reshape(1, H), w2r, w2b)


def _tc4_body(agg_ref, hr_ref, b2_ref, out_ref):
    agg = agg_ref[0, :, :C] + agg_ref[1, :, :C]
    z = agg + hr_ref[...] + b2_ref[...]
    z = z - jnp.max(z, axis=1, keepdims=True)
    ez = jnp.exp(z)
    out_ref[...] = ez / jnp.sum(ez, axis=1, keepdims=True)


def _tc4(aggC, hr, bias2):
    return pl.pallas_call(
        _tc4_body,
        grid=(N_BLKS,),
        in_specs=[
            pl.BlockSpec((NC, ROWS_BLK, CPAD), lambda i: (0, i, 0)),
            pl.BlockSpec((ROWS_BLK, C), lambda i: (i, 0)),
            pl.BlockSpec((1, C), lambda i: (0, 0)),
        ],
        out_specs=pl.BlockSpec((ROWS_BLK, C), lambda i: (i, 0)),
        out_shape=jax.ShapeDtypeStruct((N_G2, C), jnp.float32),
    )(aggC, hr, bias2.reshape(1, C))


def _sc_pass_c(hb2r_flat, dstp, gsrcp, normp):
    stripe = N2P // NS
    zrows = stripe // 4

    @functools.partial(
        pl.kernel,
        out_type=jax.ShapeDtypeStruct((NC, N2P, CPAD), jnp.float32),
        mesh=_mesh(),
        scratch_types=[
            pltpu.VMEM_SHARED((N2P, CPAD), jnp.float32),
            pltpu.VMEM((EG,), jnp.int32),
            pltpu.VMEM((EG,), jnp.int32),
            pltpu.VMEM((EG,), jnp.float32),
            pltpu.VMEM((EG, CPAD), jnp.float32),
            pltpu.VMEM((EG, CPAD), jnp.float32),
            pltpu.VMEM((zrows, CPAD), jnp.float32),
            pltpu.SemaphoreType.DMA,
        ],
        compiler_params=pltpu.CompilerParams(use_tc_tiling_on_sc=False),
    )
    def k(hb_h, dst_h, gsrc_h, norm_h, out_h,
          acc, dstbuf, gsrcbuf, normbuf, rowbuf, msgbuf, zbuf, sem):
        cid = lax.axis_index("c")
        sid = lax.axis_index("s")
        tbase = (cid * NS + sid) * TILE_E

        zv = jnp.zeros((L,), jnp.float32)

        def zrow(i, _):
            zbuf[i] = zv
            return 0
        lax.fori_loop(0, zrows, zrow, 0)
        for j in range(4):
            pltpu.sync_copy(zbuf, acc.at[pl.ds(sid * stripe + j * zrows,
                                               zrows)])
        plsc.subcore_barrier()

        def blk(b, _):
            base = tbase + b * EG
            pltpu.sync_copy(dst_h.at[pl.ds(base, EG)], dstbuf)
            pltpu.sync_copy(gsrc_h.at[pl.ds(base, EG)], gsrcbuf)
            pltpu.sync_copy(norm_h.at[pl.ds(base, EG)], normbuf)
            pltpu.async_copy(hb_h.at[gsrcbuf], rowbuf, sem).wait()
            for g in range(EG // L):
                nv = normbuf[pl.ds(g * L, L)]
                for e in range(L):
                    bc = nv[jnp.full((L,), e, jnp.int32)]
                    msgbuf[g * L + e] = rowbuf[g * L + e] * bc
            pltpu.sync_copy(msgbuf, acc.at[dstbuf], add=True)
            return 0
        lax.fori_loop(0, EBLKS, blk, 0)
        plsc.subcore_barrier()
        pltpu.sync_copy(acc.at[pl.ds(sid * stripe, stripe)],
                        out_h.at[cid, pl.ds(sid * stripe, stripe)])

    return k(hb2r_flat, dstp, gsrcp, normp)


def _jnp_counts(dst2, rel):
    keyid = dst2 * R + rel
    cnt = jax.ops.segment_sum(jnp.ones(E, jnp.float32), keyid,
                              num_segments=N_G2 * R)
    return cnt, jnp.zeros_like(cnt)


def _jnp_deg_agg_g1(src1, dst1, emb):
    m = dst1 < N_G2
    sdst = jnp.where(m, dst1, 0)
    deg = jax.ops.segment_sum(jnp.where(m, 1.0, 0.0), sdst, num_segments=N_G2)
    agg = jax.ops.segment_sum(jnp.where(m[:, None], emb[src1], 0.0), sdst,
                              num_segments=N_G2)
    return agg, deg


def _jnp_edge_w(invn, keyid, rel, comp1, comp2):
    nrm = invn[keyid][:, None]
    return nrm * comp1[rel], nrm * comp2[rel]


def _jnp_edge_pass(table, w, src, dst, width):
    rows = table[src].reshape(E, B, width)
    msg = jnp.einsum('eb,ebh->eh', w, rows)
    agg = jax.ops.segment_sum(msg, dst, num_segments=N_G2)
    return agg, jnp.zeros_like(agg)


def kernel(edge_index_g2, edge_type_g2, edge_index_g1, all_node_embedding,
           basis1, comp1, root1, bias1, basis2, comp2, root2, bias2):
    src1 = edge_index_g1[0]
    dst1 = edge_index_g1[1]
    src2 = edge_index_g2[0]
    dst2 = edge_index_g2[1]
    rel = edge_type_g2
    keyid = dst2 * R + rel

    w1a = basis1.transpose(1, 0, 2).reshape(D, B * H)
    w2r = jnp.einsum('rb,bho->rho', comp2, basis2)
    w2r = jnp.pad(w2r, ((0, 0), (0, 0), (0, CPAD - C)))

    cnt_a, cnt_b = _jnp_counts(dst2, rel)
    invn = _tc1_invnorm(cnt_a, cnt_b)

    aggA, deg = _jnp_deg_agg_g1(src1, dst1, all_node_embedding)
    w1, _ = _jnp_edge_w(invn, keyid, rel, comp1, comp2)

    padn = E_PAD - E
    dstp = jnp.concatenate([dst2, jnp.zeros((padn,), jnp.int32)])
    gsrcp = jnp.concatenate([rel * N_G2 + src2, jnp.zeros((padn,), jnp.int32)])
    normp = jnp.concatenate([invn[keyid], jnp.zeros((padn,), jnp.float32)])

    hb1, xr = _tc2(aggA, deg, w1a, root1)
    aggB_a, aggB_b = _jnp_edge_pass(hb1, w1, src2, dst2, H)
    hb2r, hr = _tc3(aggB_a + aggB_b, xr, bias1, w2r, root2)
    aggC = _sc_pass_c(hb2r.reshape(R * N_G2, CPAD), dstp, gsrcp, normp)
    return _tc4(aggC, hr, bias2)


# speedup vs baseline: 1.5885x; 1.1283x over previous
import functools

import jax
import jax.numpy as jnp
from jax import lax
from jax.experimental import pallas as pl
from jax.experimental.pallas import tpu as pltpu
from jax.experimental.pallas import tpu_sc as plsc

N_G1 = 110000
N_G2 = 100000
D = 128
H = 64
C = 8
R = 8
B = 4
E = 320000

ROWS_BLK = 2000
N_BLKS = N_G2 // ROWS_BLK

NC = 2
NS = 16
L = 16
EG = 128
E_PAD = 327680
TILE_E = E_PAD // (NC * NS)
EBLKS = TILE_E // EG
CPAD = 16
N2P = 100352


def _mesh():
    return plsc.VectorSubcoreMesh(core_axis_name="c", subcore_axis_name="s",
                                  num_cores=NC, num_subcores=NS)


def _tc1_body(cnt_a_ref, cnt_b_ref, invn_ref):
    cnt = cnt_a_ref[...] + cnt_b_ref[...]
    invn_ref[...] = 1.0 / jnp.maximum(cnt, 1.0)


def _tc1_invnorm(cnt_a, cnt_b):
    shaped = (N_G2 * R // 128, 128)
    return pl.pallas_call(
        _tc1_body,
        out_shape=jax.ShapeDtypeStruct(shaped, jnp.float32),
    )(cnt_a.reshape(shaped), cnt_b.reshape(shaped)).reshape(N_G2 * R)


def _tc2_body(agg_ref, deg_ref, w1r_ref, w1b_ref, hb1r_ref, xr_ref):
    x = jnp.maximum(agg_ref[...] / jnp.maximum(deg_ref[...], 1.0), 0.0)
    for r in range(R):
        hb1r_ref[r] = jnp.dot(x, w1r_ref[r],
                              preferred_element_type=jnp.float32)
    xr_ref[...] = jnp.dot(x, w1b_ref[...], preferred_element_type=jnp.float32)


def _tc2(aggA, deg, w1r, w1b):
    return pl.pallas_call(
        _tc2_body,
        grid=(N_BLKS,),
        in_specs=[
            pl.BlockSpec((ROWS_BLK, D), lambda i: (i, 0)),
            pl.BlockSpec((ROWS_BLK, 1), lambda i: (i, 0)),
            pl.BlockSpec((R, D, H), lambda i: (0, 0, 0)),
            pl.BlockSpec((D, H), lambda i: (0, 0)),
        ],
        out_specs=[
            pl.BlockSpec((R, ROWS_BLK, H), lambda i: (0, i, 0)),
            pl.BlockSpec((ROWS_BLK, H), lambda i: (i, 0)),
        ],
        out_shape=[
            jax.ShapeDtypeStruct((R, N_G2, H), jnp.float32),
            jax.ShapeDtypeStruct((N_G2, H), jnp.float32),
        ],
    )(aggA, deg.reshape(N_G2, 1), w1r, w1b)


def _tc3_body(agg_ref, xr_ref, b1_ref, w2r_ref, w2b_ref, hb2r_ref, hr_ref):
    h = jnp.maximum(agg_ref[...] + xr_ref[...] + b1_ref[...], 0.0)
    for r in range(R):
        hb2r_ref[r] = jnp.dot(h, w2r_ref[r],
                              preferred_element_type=jnp.float32)
    hr_ref[...] = jnp.dot(h, w2b_ref[...], preferred_element_type=jnp.float32)


def _tc3(aggB, xr, bias1, w2r, w2b):
    return pl.pallas_call(
        _tc3_body,
        grid=(N_BLKS,),
        in_specs=[
            pl.BlockSpec((ROWS_BLK, H), lambda i: (i, 0)),
            pl.BlockSpec((ROWS_BLK, H), lambda i: (i, 0)),
            pl.BlockSpec((1, H), lambda i: (0, 0)),
            pl.BlockSpec((R, H, CPAD), lambda i: (0, 0, 0)),
            pl.BlockSpec((H, C), lambda i: (0, 0)),
        ],
        out_specs=[
            pl.BlockSpec((R, ROWS_BLK, CPAD), lambda i: (0, i, 0)),
            pl.BlockSpec((ROWS_BLK, C), lambda i: (i, 0)),
        ],
        out_shape=[
            jax.ShapeDtypeStruct((R, N_G2, CPAD), jnp.float32),
            jax.ShapeDtypeStruct((N_G2, C), jnp.float32),
        ],
    )(aggB, xr, bias1.reshape(1, H), w2r, w2b)


def _tc4_body(agg_ref, hr_ref, b2_ref, out_ref):
    agg = agg_ref[0, :, :C] + agg_ref[1, :, :C]
    z = agg + hr_ref[...] + b2_ref[...]
    z = z - jnp.max(z, axis=1, keepdims=True)
    ez = jnp.exp(z)
    out_ref[...] = ez / jnp.sum(ez, axis=1, keepdims=True)


def _tc4(aggC, hr, bias2):
    return pl.pallas_call(
        _tc4_body,
        grid=(N_BLKS,),
        in_specs=[
            pl.BlockSpec((NC, ROWS_BLK, CPAD), lambda i: (0, i, 0)),
            pl.BlockSpec((ROWS_BLK, C), lambda i: (i, 0)),
            pl.BlockSpec((1, C), lambda i: (0, 0)),
        ],
        out_specs=pl.BlockSpec((ROWS_BLK, C), lambda i: (i, 0)),
        out_shape=jax.ShapeDtypeStruct((N_G2, C), jnp.float32),
    )(aggC, hr, bias2.reshape(1, C))


CH = N2P // 8
KPC = 4
BSTRIPE = CH // NS
NBLK_B = E_PAD // NS // EG


def _sc_pass_b(hb1r_flat, dstp, gsrcp, normp):
    iota16 = None

    @functools.partial(
        pl.kernel,
        out_type=jax.ShapeDtypeStruct((N2P, H), jnp.float32),
        mesh=_mesh(),
        scratch_types=[
            pltpu.VMEM_SHARED((CH + 8, H), jnp.float32),
            pltpu.VMEM((EG,), jnp.int32),
            pltpu.VMEM((EG,), jnp.int32),
            pltpu.VMEM((EG,), jnp.float32),
            pltpu.VMEM((2 * EG,), jnp.int32),
            pltpu.VMEM((2 * EG,), jnp.int32),
            pltpu.VMEM((2 * EG,), jnp.float32),
            pltpu.VMEM((EG,), jnp.int32),
            pltpu.VMEM((EG,), jnp.int32),
            pltpu.VMEM((EG,), jnp.float32),
            pltpu.VMEM((EG, H), jnp.float32),
            pltpu.VMEM((EG, H), jnp.float32),
            pltpu.VMEM((BSTRIPE, H), jnp.float32),
            pltpu.SemaphoreType.DMA,
        ],
        compiler_params=pltpu.CompilerParams(use_tc_tiling_on_sc=False,
                                             needs_layout_passes=False),
    )
    def k(hb_h, dst_h, gsrc_h, norm_h, out_h,
          acc, dstbuf, gsrcbuf, normbuf, cgsrc, cdst, cnorm,
          fgsrc, fdst, fnorm, rowbuf, msgbuf, zbuf, sem):
        cid = lax.axis_index("c")
        sid = lax.axis_index("s")
        zv = jnp.zeros((L,), jnp.float32)

        def zrow(i, _):
            for q in range(H // L):
                zbuf[i, pl.ds(q * L, L)] = zv
            return 0
        lax.fori_loop(0, BSTRIPE, zrow, 0)

        def flush_batch():
            pltpu.async_copy(hb_h.at[fgsrc], rowbuf, sem).wait()
            for g in range(EG // L):
                nv = fnorm[pl.ds(g * L, L)]
                for e in range(L):
                    bc = nv[jnp.full((L,), e, jnp.int32)]
                    for q in range(H // L):
                        msgbuf[g * L + e, pl.ds(q * L, L)] = (
                            rowbuf[g * L + e, pl.ds(q * L, L)] * bc)
            pltpu.sync_copy(msgbuf, acc.at[fdst], add=True)

        def chunk_body(kk, _):
            chunk = cid * KPC + kk
            lo = chunk * CH
            pltpu.sync_copy(zbuf, acc.at[pl.ds(sid * BSTRIPE, BSTRIPE)])

            @pl.when(sid == 0)
            def _():
                pltpu.sync_copy(zbuf.at[pl.ds(0, 8)], acc.at[pl.ds(CH, 8)])
            plsc.subcore_barrier()

            def blk(b, off):
                base = sid * (E_PAD // NS) + b * EG
                pltpu.sync_copy(dst_h.at[pl.ds(base, EG)], dstbuf)
                pltpu.sync_copy(gsrc_h.at[pl.ds(base, EG)], gsrcbuf)
                pltpu.sync_copy(norm_h.at[pl.ds(base, EG)], normbuf)
                for g in range(EG // L):
                    dstv = dstbuf[pl.ds(g * L, L)]
                    gsrcv = gsrcbuf[pl.ds(g * L, L)]
                    normv = normbuf[pl.ds(g * L, L)]
                    inr = (dstv >= lo) & (dstv < lo + CH)
                    ii = jnp.where(inr, 1, 0)
                    pos = off + plsc.cumsum(ii) - ii
                    plsc.store_scatter(cgsrc, [pos], gsrcv, mask=inr)
                    plsc.store_scatter(cdst, [pos], dstv - lo, mask=inr)
                    plsc.store_scatter(cnorm, [pos], normv, mask=inr)
                    off = off + jnp.sum(ii)

                @pl.when(off >= EG)
                def _():
                    for j in range(EG // L):
                        fgsrc[pl.ds(j * L, L)] = cgsrc[pl.ds(j * L, L)]
                        fdst[pl.ds(j * L, L)] = cdst[pl.ds(j * L, L)]
                        fnorm[pl.ds(j * L, L)] = cnorm[pl.ds(j * L, L)]
                    flush_batch()
                    for j in range(EG // L):
                        cgsrc[pl.ds(j * L, L)] = cgsrc[pl.ds(EG + j * L, L)]
                        cdst[pl.ds(j * L, L)] = cdst[pl.ds(EG + j * L, L)]
                        cnorm[pl.ds(j * L, L)] = cnorm[pl.ds(EG + j * L, L)]
                return jnp.where(off >= EG, off - EG, off)

            off = lax.fori_loop(0, NBLK_B, blk, 0)

            it = lax.iota(jnp.int32, L)
            for j in range(EG // L):
                idx = it + j * L
                m = idx < off
                fgsrc[pl.ds(j * L, L)] = jnp.where(m, cgsrc[pl.ds(j * L, L)],
                                                   0)
                fdst[pl.ds(j * L, L)] = jnp.where(m, cdst[pl.ds(j * L, L)],
                                                  CH)
                fnorm[pl.ds(j * L, L)] = jnp.where(m, cnorm[pl.ds(j * L, L)],
                                                   0.0)
            flush_batch()
            plsc.subcore_barrier()
            pltpu.sync_copy(acc.at[pl.ds(sid * BSTRIPE, BSTRIPE)],
                            out_h.at[pl.ds(lo + sid * BSTRIPE, BSTRIPE)])
            plsc.subcore_barrier()
            return 0

        lax.fori_loop(0, KPC, chunk_body, 0)

    return k(hb1r_flat, dstp, gsrcp, normp)

def _sc_pass_c(hb2r_flat, dstp, gsrcp, normp):
    stripe = N2P // NS
    zrows = stripe // 4

    @functools.partial(
        pl.kernel,
        out_type=jax.ShapeDtypeStruct((NC, N2P, CPAD), jnp.float32),
        mesh=_mesh(),
        scratch_types=[
            pltpu.VMEM_SHARED((N2P, CPAD), jnp.float32),
            pltpu.VMEM((EG,), jnp.int32),
            pltpu.VMEM((EG,), jnp.int32),
            pltpu.VMEM((EG,), jnp.float32),
            pltpu.VMEM((EG, CPAD), jnp.float32),
            pltpu.VMEM((EG, CPAD), jnp.float32),
            pltpu.VMEM((zrows, CPAD), jnp.float32),
            pltpu.SemaphoreType.DMA,
        ],
        compiler_params=pltpu.CompilerParams(use_tc_tiling_on_sc=False,
                                             needs_layout_passes=False),
    )
    def k(hb_h, dst_h, gsrc_h, norm_h, out_h,
          acc, dstbuf, gsrcbuf, normbuf, rowbuf, msgbuf, zbuf, sem):
        cid = lax.axis_index("c")
        sid = lax.axis_index("s")
        tbase = (cid * NS + sid) * TILE_E

        zv = jnp.zeros((L,), jnp.float32)

        def zrow(i, _):
            zbuf[i] = zv
            return 0
        lax.fori_loop(0, zrows, zrow, 0)
        for j in range(4):
            pltpu.sync_copy(zbuf, acc.at[pl.ds(sid * stripe + j * zrows,
                                               zrows)])
        plsc.subcore_barrier()

        def blk(b, _):
            base = tbase + b * EG
            pltpu.sync_copy(dst_h.at[pl.ds(base, EG)], dstbuf)
            pltpu.sync_copy(gsrc_h.at[pl.ds(base, EG)], gsrcbuf)
            pltpu.sync_copy(norm_h.at[pl.ds(base, EG)], normbuf)
            pltpu.async_copy(hb_h.at[gsrcbuf], rowbuf, sem).wait()
            for g in range(EG // L):
                nv = normbuf[pl.ds(g * L, L)]
                for e in range(L):
                    bc = nv[jnp.full((L,), e, jnp.int32)]
                    msgbuf[g * L + e] = rowbuf[g * L + e] * bc
            pltpu.sync_copy(msgbuf, acc.at[dstbuf], add=True)
            return 0
        lax.fori_loop(0, EBLKS, blk, 0)
        plsc.subcore_barrier()
        pltpu.sync_copy(acc.at[pl.ds(sid * stripe, stripe)],
                        out_h.at[cid, pl.ds(sid * stripe, stripe)])

    return k(hb2r_flat, dstp, gsrcp, normp)


def _jnp_counts(dst2, rel):
    keyid = dst2 * R + rel
    cnt = jax.ops.segment_sum(jnp.ones(E, jnp.float32), keyid,
                              num_segments=N_G2 * R)
    return cnt, jnp.zeros_like(cnt)


def _jnp_deg_agg_g1(src1, dst1, emb):
    m = dst1 < N_G2
    sdst = jnp.where(m, dst1, 0)
    deg = jax.ops.segment_sum(jnp.where(m, 1.0, 0.0), sdst, num_segments=N_G2)
    agg = jax.ops.segment_sum(jnp.where(m[:, None], emb[src1], 0.0), sdst,
                              num_segments=N_G2)
    return agg, deg


def _jnp_edge_w(invn, keyid, rel, comp1, comp2):
    nrm = invn[keyid][:, None]
    return nrm * comp1[rel], nrm * comp2[rel]


def _jnp_edge_pass(table, w, src, dst, width):
    rows = table[src].reshape(E, B, width)
    msg = jnp.einsum('eb,ebh->eh', w, rows)
    agg = jax.ops.segment_sum(msg, dst, num_segments=N_G2)
    return agg, jnp.zeros_like(agg)


def kernel(edge_index_g2, edge_type_g2, edge_index_g1, all_node_embedding,
           basis1, comp1, root1, bias1, basis2, comp2, root2, bias2):
    src1 = edge_index_g1[0]
    dst1 = edge_index_g1[1]
    src2 = edge_index_g2[0]
    dst2 = edge_index_g2[1]
    rel = edge_type_g2
    keyid = dst2 * R + rel

    w1r = jnp.einsum('rb,bio->rio', comp1, basis1)
    w2r = jnp.einsum('rb,bho->rho', comp2, basis2)
    w2r = jnp.pad(w2r, ((0, 0), (0, 0), (0, CPAD - C)))

    cnt_a, cnt_b = _jnp_counts(dst2, rel)
    invn = _tc1_invnorm(cnt_a, cnt_b)

    aggA, deg = _jnp_deg_agg_g1(src1, dst1, all_node_embedding)

    padn = E_PAD - E
    dstp = jnp.concatenate([dst2, jnp.zeros((padn,), jnp.int32)])
    gsrcp = jnp.concatenate([rel * N_G2 + src2, jnp.zeros((padn,), jnp.int32)])
    normp = jnp.concatenate([invn[keyid], jnp.zeros((padn,), jnp.float32)])

    hb1r, xr = _tc2(aggA, deg, w1r, root1)
    aggB = _sc_pass_b(hb1r.reshape(R * N_G2, H), dstp, gsrcp, normp)
    hb2r, hr = _tc3(aggB, xr, bias1, w2r, root2)
    aggC = _sc_pass_c(hb2r.reshape(R * N_G2, CPAD), dstp, gsrcp, normp)
    return _tc4(aggC, hr, bias2)
